# Initial kernel scaffold; baseline (speedup 1.0000x reference)
import functools

import jax
import jax.numpy as jnp
from jax import lax
from jax.experimental import pallas as pl
from jax.experimental.pallas import tpu as pltpu
from jax.experimental.pallas import tpu_sc as plsc

N = 10000
E = 160000
D = 256
H = 128
EPS = 1e-5
NC = 2
NS = 16
ET = E // NS
CH = 80
NCHUNK = ET // CH
RPT = N // NS


def _sc_mesh():
    return plsc.VectorSubcoreMesh(core_axis_name="c", subcore_axis_name="s")


@functools.partial(
    pl.kernel,
    out_type=jax.ShapeDtypeStruct((NC, N, 8), jnp.float32),
    mesh=_sc_mesh(),
    scratch_types=[
        pltpu.VMEM((CH,), jnp.int32),
        pltpu.VMEM((CH, 8), jnp.float32),
        pltpu.VMEM_SHARED((N, 8), jnp.float32),
    ],
)
def _deg_kernel(ei_hbm, ones_hbm, zeros_hbm, out_hbm, idx_v, ones_v, acc_sh):
    c = lax.axis_index("c")
    s = lax.axis_index("s")
    pltpu.sync_copy(zeros_hbm, acc_sh.at[pl.ds(s * RPT, RPT)])
    pltpu.sync_copy(ones_hbm, ones_v)
    plsc.subcore_barrier()

    def body(j, carry):
        off = s * ET + j * CH
        pltpu.sync_copy(ei_hbm.at[c, pl.ds(off, CH)], idx_v)
        pltpu.sync_copy(ones_v, acc_sh.at[idx_v], add=True)
        return carry

    lax.fori_loop(0, NCHUNK, body, 0)
    plsc.subcore_barrier()
    pltpu.sync_copy(acc_sh.at[pl.ds(s * RPT, RPT)],
                    out_hbm.at[c, pl.ds(s * RPT, RPT)])


@functools.partial(
    pl.kernel,
    out_type=jax.ShapeDtypeStruct((NC * N, H), jnp.float32),
    mesh=_sc_mesh(),
    scratch_types=[
        pltpu.VMEM((CH,), jnp.int32),
        pltpu.VMEM((CH,), jnp.int32),
        pltpu.VMEM((CH, H), jnp.float32),
        pltpu.VMEM_SHARED((N, H), jnp.float32),
        pltpu.SemaphoreType.DMA,
    ],
)
def _agg_kernel(h_hbm, ei_hbm, zeros_hbm, out_hbm, src_v, dst_v, rows_v,
                acc_sh, sem):
    c = lax.axis_index("c")
    s = lax.axis_index("s")
    pltpu.sync_copy(zeros_hbm, acc_sh.at[pl.ds(s * RPT, RPT)])
    plsc.subcore_barrier()
    col_off = c * N

    def body(j, carry):
        off = s * ET + j * CH
        pltpu.sync_copy(ei_hbm.at[0, pl.ds(off, CH)], src_v)
        pltpu.sync_copy(ei_hbm.at[1, pl.ds(off, CH)], dst_v)
        for i in range(CH // 16):
            sl = pl.ds(i * 16, 16)
            src_v[sl] = src_v[sl] + col_off
        pltpu.async_copy(h_hbm.at[src_v], rows_v, sem).wait()
        pltpu.sync_copy(rows_v, acc_sh.at[dst_v], add=True)
        return carry

    lax.fori_loop(0, NCHUNK, body, 0)
    plsc.subcore_barrier()
    pltpu.sync_copy(acc_sh.at[pl.ds(s * RPT, RPT)],
                    out_hbm.at[pl.ds(c * N + s * RPT, RPT)])


def _prep_body(feats_ref, deg_ref, out_ref):
    ns = lax.rsqrt(jnp.maximum(deg_ref[...], 1.0))
    h = feats_ref[...] * ns
    out_ref[pl.ds(0, N), :] = h[:, :H]
    out_ref[pl.ds(N, N), :] = h[:, H:]


_prep = pl.pallas_call(
    _prep_body,
    out_shape=jax.ShapeDtypeStruct((NC * N, H), jnp.float32),
)


def _dense_body(last, a_ref, indeg_ref, outdeg_ref, w_ref, b_ref, g_ref,
                be_ref, al_ref, out_ref):
    a0 = a_ref[0]
    a1 = a_ref[1]
    z = jnp.dot(a0, w_ref[0:H, :], preferred_element_type=jnp.float32)
    z = z + jnp.dot(a1, w_ref[H:D, :], preferred_element_type=jnp.float32)
    nd = lax.rsqrt(jnp.maximum(indeg_ref[...], 1.0))
    z = z * nd + b_ref[...]
    mean = jnp.mean(z, axis=0, keepdims=True)
    zc = z - mean
    var = jnp.mean(zc * zc, axis=0, keepdims=True)
    y = zc * lax.rsqrt(var + EPS) * g_ref[...] + be_ref[...]
    y = jnp.where(y >= 0, y, al_ref[0, 0] * y)
    if last:
        out_ref[...] = y
    else:
        ns = lax.rsqrt(jnp.maximum(outdeg_ref[...], 1.0))
        hn = y * ns
        out_ref[pl.ds(0, N), :] = hn[:, :H]
        out_ref[pl.ds(N, N), :] = hn[:, H:]


_dense_mid = pl.pallas_call(
    functools.partial(_dense_body, False),
    out_shape=jax.ShapeDtypeStruct((NC * N, H), jnp.float32),
)
_dense_last = pl.pallas_call(
    functools.partial(_dense_body, True),
    out_shape=jax.ShapeDtypeStruct((N, D), jnp.float32),
)


def kernel(feats, edge_index, W1, b1, gamma1, beta1, a1,
           W2, b2, gamma2, beta2, a2):
    ei = edge_index.astype(jnp.int32)
    ones8 = jnp.zeros((CH, 8), jnp.float32).at[:, 0].set(1.0)
    zeros8 = jnp.zeros((RPT, 8), jnp.float32)
    zeros_h = jnp.zeros((RPT, H), jnp.float32)

    degs = _deg_kernel(ei, ones8, zeros8)
    out_deg = degs[0, :, 0:1]
    in_deg = degs[1, :, 0:1]

    b1r, g1r, be1r = b1.reshape(1, D), gamma1.reshape(1, D), beta1.reshape(1, D)
    b2r, g2r, be2r = b2.reshape(1, D), gamma2.reshape(1, D), beta2.reshape(1, D)
    a1r, a2r = a1.reshape(1, 1), a2.reshape(1, 1)

    h1 = _prep(feats, out_deg)
    agg1 = _agg_kernel(h1, ei, zeros_h)
    h2 = _dense_mid(agg1.reshape(NC, N, H), in_deg, out_deg,
                    W1, b1r, g1r, be1r, a1r)
    agg2 = _agg_kernel(h2, ei, zeros_h)
    out = _dense_last(agg2.reshape(NC, N, H), in_deg, out_deg,
                      W2, b2r, g2r, be2r, a2r)
    return out


# speedup vs baseline: 3.4265x; 3.4265x over previous
import functools

import jax
import jax.numpy as jnp
from jax import lax
from jax.experimental import pallas as pl
from jax.experimental.pallas import tpu as pltpu
from jax.experimental.pallas import tpu_sc as plsc

N = 10000
NP = 10240
E = 160000
D = 256
H = 128
EPS = 1e-5
NC = 2
NS = 16
ET = E // NS
CH = 80
NCHUNK = ET // CH
RPT = NP // NS


def _sc_mesh():
    return plsc.VectorSubcoreMesh(core_axis_name="c", subcore_axis_name="s")


ETD = E // (NC * NS)
NVREG = ETD // 16
REM = ETD - NVREG * 16


@functools.partial(
    pl.kernel,
    out_type=jax.ShapeDtypeStruct((NC * 2 * NP,), jnp.float32),
    mesh=_sc_mesh(),
    compiler_params=pltpu.CompilerParams(needs_layout_passes=False),
    scratch_types=[
        pltpu.VMEM((ETD + 16,), jnp.int32),
        pltpu.VMEM((NP,), jnp.float32),
        pltpu.VMEM((NS, RPT), jnp.float32),
        pltpu.VMEM_SHARED((NS, NP), jnp.float32),
        pltpu.VMEM_SHARED((NS, NP), jnp.float32),
    ],
)
def _deg_kernel(src_hbm, dst_hbm, out_hbm, idx_v, hist_v, red_v,
                stage_s, stage_d):
    c = lax.axis_index("c")
    s = lax.axis_index("s")
    base = c * (E // NC) + s * ETD
    ones = jnp.full((16,), 1.0, jnp.float32)
    zv = jnp.zeros((16,), jnp.float32)
    rem_mask = lax.iota(jnp.int32, 16) < REM

    def histogram(edge_hbm, stage):
        def zero_body(j, carry):
            hist_v[pl.ds(j * 16, 16)] = zv
            return carry
        lax.fori_loop(0, NP // 16, zero_body, 0)
        pltpu.sync_copy(edge_hbm.at[pl.ds(base, ETD)], idx_v.at[pl.ds(0, ETD)])

        def body(j, carry):
            idx = idx_v[pl.ds(j * 16, 16)]
            plsc.addupdate_scatter(hist_v, [idx], ones)
            return carry
        lax.fori_loop(0, NVREG, body, 0)
        if REM:
            idx = idx_v[pl.ds(NVREG * 16, 16)]
            plsc.addupdate_scatter(hist_v, [idx], ones, mask=rem_mask)
        pltpu.sync_copy(hist_v, stage.at[s])

    def reduce_out(stage, t):
        pltpu.sync_copy(stage.at[:, pl.ds(s * RPT, RPT)], red_v)

        def red_body(g, carry):
            acc = jnp.zeros((16,), jnp.float32)
            for r in range(NS):
                acc = acc + red_v[r, pl.ds(g * 16, 16)]
            hist_v[pl.ds(g * 16, 16)] = acc
            return carry
        lax.fori_loop(0, RPT // 16, red_body, 0)
        pltpu.sync_copy(hist_v.at[pl.ds(0, RPT)],
                        out_hbm.at[pl.ds(c * (2 * NP) + t * NP + s * RPT, RPT)])

    histogram(src_hbm, stage_s)
    histogram(dst_hbm, stage_d)
    plsc.subcore_barrier()
    reduce_out(stage_s, 0)
    reduce_out(stage_d, 1)


@functools.partial(
    pl.kernel,
    out_type=jax.ShapeDtypeStruct((NC * NP, H), jnp.float32),
    mesh=_sc_mesh(),
    scratch_types=[
        pltpu.VMEM((CH,), jnp.int32),
        pltpu.VMEM((CH,), jnp.int32),
        pltpu.VMEM((CH, H), jnp.float32),
        pltpu.VMEM_SHARED((NP, H), jnp.float32),
        pltpu.SemaphoreType.DMA,
    ],
)
def _agg_kernel(h_hbm, src_hbm, dst_hbm, zeros_hbm, out_hbm,
                src_v, dst_v, rows_v, acc_sh, sem):
    c = lax.axis_index("c")
    s = lax.axis_index("s")
    pltpu.sync_copy(zeros_hbm, acc_sh.at[pl.ds(s * RPT, RPT)])
    plsc.subcore_barrier()
    col_off = c * N

    def body(j, carry):
        off = s * ET + j * CH
        pltpu.sync_copy(src_hbm.at[pl.ds(off, CH)], src_v)
        pltpu.sync_copy(dst_hbm.at[pl.ds(off, CH)], dst_v)
        for i in range(CH // 16):
            sl = pl.ds(i * 16, 16)
            src_v[sl] = src_v[sl] + col_off
        pltpu.async_copy(h_hbm.at[src_v], rows_v, sem).wait()
        pltpu.sync_copy(rows_v, acc_sh.at[dst_v], add=True)
        return carry

    lax.fori_loop(0, NCHUNK, body, 0)
    plsc.subcore_barrier()
    pltpu.sync_copy(acc_sh.at[pl.ds(s * RPT, RPT)],
                    out_hbm.at[pl.ds(c * NP + s * RPT, RPT)])


def _prep_body(feats_ref, deg_ref, out_ref):
    ns = lax.rsqrt(jnp.maximum(deg_ref[...], 1.0))
    h = feats_ref[...] * ns
    out_ref[pl.ds(0, N), :] = h[:, :H]
    out_ref[pl.ds(N, N), :] = h[:, H:]


_prep = pl.pallas_call(
    _prep_body,
    out_shape=jax.ShapeDtypeStruct((NC * N, H), jnp.float32),
)


def _dense_body(last, a_ref, indeg_ref, outdeg_ref, w_ref, b_ref, g_ref,
                be_ref, al_ref, out_ref):
    a0 = a_ref[0]
    a1 = a_ref[1]
    z = jnp.dot(a0, w_ref[0:H, :], preferred_element_type=jnp.float32)
    z = z + jnp.dot(a1, w_ref[H:D, :], preferred_element_type=jnp.float32)
    nd = lax.rsqrt(jnp.maximum(indeg_ref[...], 1.0))
    z = z * nd + b_ref[...]
    mean = jnp.mean(z, axis=0, keepdims=True)
    zc = z - mean
    var = jnp.mean(zc * zc, axis=0, keepdims=True)
    y = zc * lax.rsqrt(var + EPS) * g_ref[...] + be_ref[...]
    y = jnp.where(y >= 0, y, al_ref[0, 0] * y)
    if last:
        out_ref[...] = y
    else:
        ns = lax.rsqrt(jnp.maximum(outdeg_ref[...], 1.0))
        hn = y * ns
        out_ref[pl.ds(0, N), :] = hn[:, :H]
        out_ref[pl.ds(N, N), :] = hn[:, H:]


_dense_mid = pl.pallas_call(
    functools.partial(_dense_body, False),
    out_shape=jax.ShapeDtypeStruct((NC * N, H), jnp.float32),
)
_dense_last = pl.pallas_call(
    functools.partial(_dense_body, True),
    out_shape=jax.ShapeDtypeStruct((N, D), jnp.float32),
)


def _halves(x_padded):
    return x_padded.reshape(NC, NP, -1)[:, :N, :]


def kernel(feats, edge_index, W1, b1, gamma1, beta1, a1,
           W2, b2, gamma2, beta2, a2):
    ei = edge_index.astype(jnp.int32)
    src, dst = ei[0], ei[1]
    zeros_h = jnp.zeros((RPT, H), jnp.float32)

    degs = _deg_kernel(src, dst).reshape(NC, 2, NP)
    deg = degs[0, :, :N, None] + degs[1, :, :N, None]
    out_deg = deg[0]
    in_deg = deg[1]

    b1r, g1r, be1r = b1.reshape(1, D), gamma1.reshape(1, D), beta1.reshape(1, D)
    b2r, g2r, be2r = b2.reshape(1, D), gamma2.reshape(1, D), beta2.reshape(1, D)
    a1r, a2r = a1.reshape(1, 1), a2.reshape(1, 1)

    h1 = _prep(feats, out_deg)
    agg1 = _halves(_agg_kernel(h1, src, dst, zeros_h))
    h2 = _dense_mid(agg1, in_deg, out_deg,
                    W1, b1r, g1r, be1r, a1r)
    agg2 = _halves(_agg_kernel(h2, src, dst, zeros_h))
    out = _dense_last(agg2, in_deg, out_deg,
                      W2, b2r, g2r, be2r, a2r)
    return out


# speedup vs baseline: 6.2807x; 1.8330x over previous
import functools

import jax
import jax.numpy as jnp
from jax import lax
from jax.experimental import pallas as pl
from jax.experimental.pallas import tpu as pltpu
from jax.experimental.pallas import tpu_sc as plsc

N = 10000
NP = 10240
E = 160000
D = 256
H = 128
EPS = 1e-5
NC = 2
NS = 16
ET = E // NS
CH = 80
NCHUNK = ET // CH
NCHB = 128
RPT = NP // NS


def _sc_mesh():
    return plsc.VectorSubcoreMesh(core_axis_name="c", subcore_axis_name="s")


ETD = E // (NC * NS)
NVREG = ETD // 16
REM = ETD - NVREG * 16


@functools.partial(
    pl.kernel,
    out_type=jax.ShapeDtypeStruct((NC * 2 * NP,), jnp.float32),
    mesh=_sc_mesh(),
    compiler_params=pltpu.CompilerParams(needs_layout_passes=False),
    scratch_types=[
        pltpu.VMEM((ETD + 16,), jnp.int32),
        pltpu.VMEM((NP,), jnp.float32),
        pltpu.VMEM((NS, RPT), jnp.float32),
        pltpu.VMEM_SHARED((NS, NP), jnp.float32),
        pltpu.VMEM_SHARED((NS, NP), jnp.float32),
    ],
)
def _deg_kernel(src_hbm, dst_hbm, out_hbm, idx_v, hist_v, red_v,
                stage_s, stage_d):
    c = lax.axis_index("c")
    s = lax.axis_index("s")
    base = c * (E // NC) + s * ETD
    ones = jnp.full((16,), 1.0, jnp.float32)
    zv = jnp.zeros((16,), jnp.float32)
    rem_mask = lax.iota(jnp.int32, 16) < REM

    def histogram(edge_hbm, stage):
        def zero_body(j, carry):
            hist_v[pl.ds(j * 16, 16)] = zv
            return carry
        lax.fori_loop(0, NP // 16, zero_body, 0)
        pltpu.sync_copy(edge_hbm.at[pl.ds(base, ETD)], idx_v.at[pl.ds(0, ETD)])

        def body(j, carry):
            idx = idx_v[pl.ds(j * 16, 16)]
            plsc.addupdate_scatter(hist_v, [idx], ones)
            return carry
        lax.fori_loop(0, NVREG, body, 0)
        if REM:
            idx = idx_v[pl.ds(NVREG * 16, 16)]
            plsc.addupdate_scatter(hist_v, [idx], ones, mask=rem_mask)
        pltpu.sync_copy(hist_v, stage.at[s])

    def reduce_out(stage, t):
        pltpu.sync_copy(stage.at[:, pl.ds(s * RPT, RPT)], red_v)

        def red_body(g, carry):
            acc = jnp.zeros((16,), jnp.float32)
            for r in range(NS):
                acc = acc + red_v[r, pl.ds(g * 16, 16)]
            hist_v[pl.ds(g * 16, 16)] = acc
            return carry
        lax.fori_loop(0, RPT // 16, red_body, 0)
        pltpu.sync_copy(hist_v.at[pl.ds(0, RPT)],
                        out_hbm.at[pl.ds(c * (2 * NP) + t * NP + s * RPT, RPT)])

    histogram(src_hbm, stage_s)
    histogram(dst_hbm, stage_d)
    plsc.subcore_barrier()
    reduce_out(stage_s, 0)
    reduce_out(stage_d, 1)


@functools.partial(
    pl.kernel,
    out_type=jax.ShapeDtypeStruct((NC * NP, H), jnp.float32),
    mesh=_sc_mesh(),
    scratch_types=[
        pltpu.VMEM((CH,), jnp.int32),
        pltpu.VMEM((CH,), jnp.int32),
        pltpu.VMEM((NCHB, CH), jnp.int32),
        pltpu.VMEM((CH, H), jnp.float32),
        pltpu.VMEM((CH, H), jnp.float32),
        pltpu.VMEM_SHARED((NP, H), jnp.float32),
        pltpu.SemaphoreType.DMA,
        pltpu.SemaphoreType.DMA,
    ],
)
def _agg_kernel(h_hbm, srcf_hbm, dst2d_hbm, zeros_hbm, out_hbm,
                src0, src1, dst_v, rows0, rows1, acc_sh, sem0, sem1):
    c = lax.axis_index("c")
    s = lax.axis_index("s")
    pltpu.sync_copy(zeros_hbm, acc_sh.at[pl.ds(s * RPT, RPT)])
    pltpu.sync_copy(dst2d_hbm.at[pl.ds(s * NCHB, NCHB)], dst_v)
    plsc.subcore_barrier()
    ebase = c * E + s * ET

    def load_idx(j, buf):
        pltpu.sync_copy(srcf_hbm.at[pl.ds(ebase + j * CH, CH)], buf)

    def gather(buf, rows, sem):
        return pltpu.make_async_copy(h_hbm.at[buf], rows, sem)

    def scatter(j, rows):
        pltpu.sync_copy(rows, acc_sh.at[dst_v.at[j]], add=True)

    load_idx(0, src0)
    gather(src0, rows0, sem0).start()

    def body(p, carry):
        j0 = 2 * p
        load_idx(j0 + 1, src1)
        gather(src1, rows1, sem1).start()
        gather(src0, rows0, sem0).wait()
        scatter(j0, rows0)
        load_idx(j0 + 2, src0)
        gather(src0, rows0, sem0).start()
        gather(src1, rows1, sem1).wait()
        scatter(j0 + 1, rows1)
        return carry

    lax.fori_loop(0, (NCHUNK - 1) // 2, body, 0)
    gather(src0, rows0, sem0).wait()
    scatter(NCHUNK - 1, rows0)
    plsc.subcore_barrier()
    pltpu.sync_copy(acc_sh.at[pl.ds(s * RPT, RPT)],
                    out_hbm.at[pl.ds(c * NP + s * RPT, RPT)])


def _prep_body(feats_ref, deg_ref, out_ref):
    ns = lax.rsqrt(jnp.maximum(deg_ref[...], 1.0))
    h = feats_ref[...] * ns
    out_ref[pl.ds(0, N), :] = h[:, :H]
    out_ref[pl.ds(N, N), :] = h[:, H:]


_prep = pl.pallas_call(
    _prep_body,
    out_shape=jax.ShapeDtypeStruct((NC * N, H), jnp.float32),
)


def _dense_body(last, a_ref, indeg_ref, outdeg_ref, w_ref, b_ref, g_ref,
                be_ref, al_ref, out_ref):
    a0 = a_ref[0]
    a1 = a_ref[1]
    z = jnp.dot(a0, w_ref[0:H, :], preferred_element_type=jnp.float32)
    z = z + jnp.dot(a1, w_ref[H:D, :], preferred_element_type=jnp.float32)
    nd = lax.rsqrt(jnp.maximum(indeg_ref[...], 1.0))
    z = z * nd + b_ref[...]
    mean = jnp.mean(z, axis=0, keepdims=True)
    zc = z - mean
    var = jnp.mean(zc * zc, axis=0, keepdims=True)
    y = zc * lax.rsqrt(var + EPS) * g_ref[...] + be_ref[...]
    y = jnp.where(y >= 0, y, al_ref[0, 0] * y)
    if last:
        out_ref[...] = y
    else:
        ns = lax.rsqrt(jnp.maximum(outdeg_ref[...], 1.0))
        hn = y * ns
        out_ref[pl.ds(0, N), :] = hn[:, :H]
        out_ref[pl.ds(N, N), :] = hn[:, H:]


_dense_mid = pl.pallas_call(
    functools.partial(_dense_body, False),
    out_shape=jax.ShapeDtypeStruct((NC * N, H), jnp.float32),
)
_dense_last = pl.pallas_call(
    functools.partial(_dense_body, True),
    out_shape=jax.ShapeDtypeStruct((N, D), jnp.float32),
)


def _halves(x_padded):
    return x_padded.reshape(NC, NP, -1)[:, :N, :]


def kernel(feats, edge_index, W1, b1, gamma1, beta1, a1,
           W2, b2, gamma2, beta2, a2):
    ei = edge_index.astype(jnp.int32)
    src, dst = ei[0], ei[1]
    zeros_h = jnp.zeros((RPT, H), jnp.float32)
    srcf = jnp.concatenate([src, src + N])
    pad = ((0, 0), (0, NCHB - NCHUNK), (0, 0))
    dst2d = jnp.pad(dst.reshape(NS, NCHUNK, CH), pad).reshape(NS * NCHB, CH)

    degs = _deg_kernel(src, dst).reshape(NC, 2, NP)
    deg = degs[0, :, :N, None] + degs[1, :, :N, None]
    out_deg = deg[0]
    in_deg = deg[1]

    b1r, g1r, be1r = b1.reshape(1, D), gamma1.reshape(1, D), beta1.reshape(1, D)
    b2r, g2r, be2r = b2.reshape(1, D), gamma2.reshape(1, D), beta2.reshape(1, D)
    a1r, a2r = a1.reshape(1, 1), a2.reshape(1, 1)

    h1 = _prep(feats, out_deg)
    agg1 = _halves(_agg_kernel(h1, srcf, dst2d, zeros_h))
    h2 = _dense_mid(agg1, in_deg, out_deg,
                    W1, b1r, g1r, be1r, a1r)
    agg2 = _halves(_agg_kernel(h2, srcf, dst2d, zeros_h))
    out = _dense_last(agg2, in_deg, out_deg,
                      W2, b2r, g2r, be2r, a2r)
    return out
